# Initial kernel scaffold; baseline (speedup 1.0000x reference)
import jax
import jax.numpy as jnp
from jax.experimental import pallas as pl


def kernel(edge_index1, edge_weight1, edge_index2, edge_weight2, W1, W2, W3, W4, lw1, lb1, lw2, lb2, lw3, lb3):
    raise NotImplementedError("write your pallas kernel here")


# speedup vs baseline: 219.2440x; 219.2440x over previous
import functools

import jax
import jax.numpy as jnp
from jax import lax
from jax.experimental import pallas as pl
from jax.experimental.pallas import tpu as pltpu
from jax.experimental.pallas import tpu_sc as plsc

N = 10000
NP = 10240
E = 320000
NHID = 128

NC = 2
NS = 16
CHUNK = 80
SUP = 25
EPT = E // NS
NSUP = EPT // (CHUNK * SUP)
RPT = NP // NS


def _make_spmm():
    mesh = plsc.VectorSubcoreMesh(core_axis_name="c", subcore_axis_name="s")

    @functools.partial(
        pl.kernel,
        out_type=jax.ShapeDtypeStruct((NC, NP, NHID), jnp.float32),
        mesh=mesh,
        scratch_types=[
            pltpu.VMEM((SUP, CHUNK), jnp.int32),
            pltpu.VMEM((SUP, CHUNK), jnp.int32),
            pltpu.VMEM((SUP, CHUNK), jnp.float32),
            pltpu.VMEM((CHUNK, NHID), jnp.float32),
            pltpu.VMEM((CHUNK, NHID), jnp.float32),
            pltpu.VMEM_SHARED((NP, NHID), jnp.float32),
            pltpu.SemaphoreType.DMA,
            pltpu.SemaphoreType.DMA,
            pltpu.SemaphoreType.DMA,
            pltpu.SemaphoreType.DMA,
        ],
    )
    def spmm(src_hbm, dst_hbm, w_hbm, x_hbm, zeros_hbm, out_hbm,
             src_v, dst_v, w_v, rows_a, rows_b, acc,
             sem_ga, sem_gb, sem_sa, sem_sb):
        i32 = jnp.int32
        c = lax.axis_index("c").astype(i32)
        s = lax.axis_index("s").astype(i32)

        pltpu.sync_copy(zeros_hbm, acc.at[pl.ds(s * i32(RPT), RPT)])
        plsc.subcore_barrier()

        def gather_start(j, buf, sem):
            pltpu.async_copy(x_hbm.at[c].at[src_v.at[j]], buf, sem)

        def gather_wait(j, buf, sem):
            pltpu.make_async_copy(x_hbm.at[c].at[src_v.at[j]], buf, sem).wait()

        def scatter_start(j, buf, sem):
            pltpu.async_copy(buf, acc.at[dst_v.at[j]], sem, add=True)

        def scatter_wait(j, buf, sem):
            pltpu.make_async_copy(buf, acc.at[dst_v.at[j]], sem).wait()

        def scale(j, buf):
            def scale16(g, carry2):
                w16 = w_v[j, pl.ds(g * i32(16), 16)]
                for r in range(16):
                    wr = w16[r]
                    row = g * i32(16) + i32(r)
                    for cj in range(NHID // 16):
                        sl = pl.ds(cj * 16, 16)
                        buf[row, sl] = buf[row, sl] * wr
                return carry2

            lax.fori_loop(i32(0), i32(CHUNK // 16), scale16, i32(0))

        def superchunk(u, carry0):
            pltpu.sync_copy(src_hbm.at[c, s, u], src_v)
            pltpu.sync_copy(dst_hbm.at[c, s, u], dst_v)
            pltpu.sync_copy(w_hbm.at[c, s, u], w_v)

            gather_start(i32(0), rows_a, sem_ga)

            def pair(p, carry):
                a = p * i32(2)

                @pl.when(p > i32(0))
                def _():
                    scatter_wait(a - i32(1), rows_b, sem_sb)

                gather_start(a + i32(1), rows_b, sem_gb)
                gather_wait(a, rows_a, sem_ga)
                scale(a, rows_a)
                scatter_start(a, rows_a, sem_sa)
                gather_wait(a + i32(1), rows_b, sem_gb)
                scale(a + i32(1), rows_b)
                scatter_wait(a, rows_a, sem_sa)
                gather_start(a + i32(2), rows_a, sem_ga)
                scatter_start(a + i32(1), rows_b, sem_sb)
                return carry

            lax.fori_loop(i32(0), i32(SUP // 2), pair, i32(0))
            last = i32(SUP - 1)
            gather_wait(last, rows_a, sem_ga)
            scale(last, rows_a)
            scatter_wait(last - i32(1), rows_b, sem_sb)
            scatter_start(last, rows_a, sem_sa)
            scatter_wait(last, rows_a, sem_sa)
            return carry0

        lax.fori_loop(i32(0), i32(NSUP), superchunk, i32(0))
        plsc.subcore_barrier()
        pltpu.sync_copy(acc.at[pl.ds(s * i32(RPT), RPT)],
                        out_hbm.at[c, pl.ds(s * i32(RPT), RPT)])

    return spmm


_spmm = _make_spmm()


BLK = 1024

_PREC = lax.Precision.HIGHEST


def _dot(a, b):
    return jnp.dot(a, b, preferred_element_type=jnp.float32, precision=_PREC)


def _tc_layer_body(p_ref, w_ref, lw1_ref, lb1_ref, lw2_ref, lb2_ref,
                   lw3_ref, lb3_ref, sacc_ref, y_ref, s_ref):
    h = p_ref[0]
    r = jnp.maximum(h, 0.0)
    nrm = jnp.sqrt(jnp.sum(r * r, axis=1, keepdims=True))
    x = r / jnp.maximum(nrm, 1e-12)
    y_ref[0] = _dot(x, w_ref[...])
    m = jnp.maximum(_dot(x, lw1_ref[...]) + lb1_ref[...], 0.0)
    m = jnp.maximum(_dot(m, lw2_ref[...]) + lb2_ref[...], 0.0)
    s_ref[0] = sacc_ref[0] + _dot(m, lw3_ref[...]) + lb3_ref[...]


def _tc_tail_body(p_ref, lw1_ref, lb1_ref, lw2_ref, lb2_ref,
                  lw3_ref, lb3_ref, sacc_ref, s_ref):
    def head(b):
        x = jnp.maximum(p_ref[b], 0.0)
        m = jnp.maximum(_dot(x, lw1_ref[...]) + lb1_ref[...], 0.0)
        m = jnp.maximum(_dot(m, lw2_ref[...]) + lb2_ref[...], 0.0)
        return sacc_ref[b] + _dot(m, lw3_ref[...]) + lb3_ref[...]

    s_ref[...] = head(0) * head(1)


def _full(shape):
    return pl.BlockSpec(shape, lambda *_: tuple(jnp.int32(0) for _ in shape))


def _tc_layer(p, w, lw1, lb1, lw2, lb2, lw3, lb3, sacc):
    return pl.pallas_call(
        _tc_layer_body,
        grid=(NC, NP // BLK),
        in_specs=[
            pl.BlockSpec((1, BLK, NHID), lambda b, i: (b, i, jnp.int32(0))),
            _full((NHID, NHID)),
            _full((NHID, 2 * NHID)), _full((1, 2 * NHID)),
            _full((2 * NHID, 2 * NHID)), _full((1, 2 * NHID)),
            _full((2 * NHID, 1)), _full((1, 1)),
            pl.BlockSpec((1, BLK, 1), lambda b, i: (b, i, jnp.int32(0))),
        ],
        out_specs=[
            pl.BlockSpec((1, BLK, NHID), lambda b, i: (b, i, jnp.int32(0))),
            pl.BlockSpec((1, BLK, 1), lambda b, i: (b, i, jnp.int32(0))),
        ],
        out_shape=[
            jax.ShapeDtypeStruct((NC, NP, NHID), jnp.float32),
            jax.ShapeDtypeStruct((NC, NP, 1), jnp.float32),
        ],
    )(p, w, lw1, lb1, lw2, lb2, lw3, lb3, sacc)


def _tc_tail(p, lw1, lb1, lw2, lb2, lw3, lb3, sacc):
    return pl.pallas_call(
        _tc_tail_body,
        grid=(NP // BLK,),
        in_specs=[
            pl.BlockSpec((NC, BLK, NHID), lambda i: (jnp.int32(0), i, jnp.int32(0))),
            _full((NHID, 2 * NHID)), _full((1, 2 * NHID)),
            _full((2 * NHID, 2 * NHID)), _full((1, 2 * NHID)),
            _full((2 * NHID, 1)), _full((1, 1)),
            pl.BlockSpec((NC, BLK, 1), lambda i: (jnp.int32(0), i, jnp.int32(0))),
        ],
        out_specs=pl.BlockSpec((BLK, 1), lambda i: (i, jnp.int32(0))),
        out_shape=jax.ShapeDtypeStruct((NP, 1), jnp.float32),
    )(p, lw1, lb1, lw2, lb2, lw3, lb3, sacc)


def kernel(edge_index1, edge_weight1, edge_index2, edge_weight2,
           W1, W2, W3, W4, lw1, lb1, lw2, lb2, lw3, lb3):
    out_dtype = jnp.result_type(edge_weight1.dtype, W1.dtype, lw1.dtype,
                                lb1.dtype, lw3.dtype)
    f32 = jnp.float32
    W1, W2, W3, W4 = (a.astype(f32) for a in (W1, W2, W3, W4))
    lw1, lw2, lw3 = (a.astype(f32) for a in (lw1, lw2, lw3))
    lb1, lb2, lb3 = (a.astype(f32) for a in (lb1, lb2, lb3))
    zeros = jnp.zeros((RPT, NHID), jnp.float32)
    lb1r = lb1.reshape(1, 2 * NHID)
    lb2r = lb2.reshape(1, 2 * NHID)
    lb3r = lb3.reshape(1, 1)
    s0 = jnp.zeros((NC, NP, 1), jnp.float32)

    def edges(ei):
        i = ei.astype(jnp.int32)
        return i.reshape(2, NS, NSUP, SUP, CHUNK)

    e1 = edges(edge_index1)
    e2 = edges(edge_index2)
    src = jnp.stack([e1[1], e2[1]])
    dst = jnp.stack([e1[0], e2[0]])
    w = jnp.stack([edge_weight1.astype(f32).reshape(NS, NSUP, SUP, CHUNK),
                   edge_weight2.astype(f32).reshape(NS, NSUP, SUP, CHUNK)])

    x = jnp.stack([W1, W1])
    p = _spmm(src, dst, w, x, zeros)
    y, s = _tc_layer(p, W2, lw1, lb1r, lw2, lb2r, lw3, lb3r, s0)
    p = _spmm(src, dst, w, y, zeros)
    y, s = _tc_layer(p, W3, lw1, lb1r, lw2, lb2r, lw3, lb3r, s)
    p = _spmm(src, dst, w, y, zeros)
    y, s = _tc_layer(p, W4, lw1, lb1r, lw2, lb2r, lw3, lb3r, s)
    p = _spmm(src, dst, w, y, zeros)
    out = _tc_tail(p, lw1, lb1r, lw2, lb2r, lw3, lb3r, s)
    return out[:N].astype(out_dtype)


# speedup vs baseline: 221.3316x; 1.0095x over previous
import functools

import jax
import jax.numpy as jnp
from jax import lax
from jax.experimental import pallas as pl
from jax.experimental.pallas import tpu as pltpu
from jax.experimental.pallas import tpu_sc as plsc

N = 10000
NP = 10240
E = 320000
NHID = 128

NC = 2
NS = 16
CHUNK = 80
SUP = 25
EPT = E // NS
NSUP = EPT // (CHUNK * SUP)
RPT = NP // NS


def _make_spmm():
    mesh = plsc.VectorSubcoreMesh(core_axis_name="c", subcore_axis_name="s")

    @functools.partial(
        pl.kernel,
        out_type=jax.ShapeDtypeStruct((NC, NP, NHID), jnp.float32),
        mesh=mesh,
        scratch_types=[
            pltpu.VMEM((SUP, CHUNK), jnp.int32),
            pltpu.VMEM((SUP, CHUNK), jnp.int32),
            pltpu.VMEM((SUP, CHUNK), jnp.float32),
            pltpu.VMEM((CHUNK, NHID), jnp.float32),
            pltpu.VMEM((CHUNK, NHID), jnp.float32),
            pltpu.VMEM_SHARED((NP, NHID), jnp.float32),
            pltpu.SemaphoreType.DMA,
            pltpu.SemaphoreType.DMA,
            pltpu.SemaphoreType.DMA,
            pltpu.SemaphoreType.DMA,
        ],
    )
    def spmm(src1_hbm, dst1_hbm, w1_hbm, src2_hbm, dst2_hbm, w2_hbm,
             x_hbm, zeros_hbm, out_hbm,
             src_v, dst_v, w_v, rows_a, rows_b, acc,
             sem_ga, sem_gb, sem_sa, sem_sb):
        i32 = jnp.int32
        c = lax.axis_index("c").astype(i32)
        s = lax.axis_index("s").astype(i32)

        pltpu.sync_copy(zeros_hbm, acc.at[pl.ds(s * i32(RPT), RPT)])
        plsc.subcore_barrier()

        def gather_start(j, buf, sem):
            pltpu.async_copy(x_hbm.at[c].at[src_v.at[j]], buf, sem)

        def gather_wait(j, buf, sem):
            pltpu.make_async_copy(x_hbm.at[c].at[src_v.at[j]], buf, sem).wait()

        def scatter_start(j, buf, sem):
            pltpu.async_copy(buf, acc.at[dst_v.at[j]], sem, add=True)

        def scatter_wait(j, buf, sem):
            pltpu.make_async_copy(buf, acc.at[dst_v.at[j]], sem).wait()

        def scale(j, buf):
            def scale16(g, carry2):
                w16 = w_v[j, pl.ds(g * i32(16), 16)]
                for r in range(16):
                    wr = w16[r]
                    row = g * i32(16) + i32(r)
                    for cj in range(NHID // 16):
                        sl = pl.ds(cj * 16, 16)
                        buf[row, sl] = buf[row, sl] * wr
                return carry2

            lax.fori_loop(i32(0), i32(CHUNK // 16), scale16, i32(0))

        def superchunk(u, carry0):
            @pl.when(c == i32(0))
            def _():
                pltpu.sync_copy(src1_hbm.at[s, u], src_v)
                pltpu.sync_copy(dst1_hbm.at[s, u], dst_v)
                pltpu.sync_copy(w1_hbm.at[s, u], w_v)

            @pl.when(c == i32(1))
            def _():
                pltpu.sync_copy(src2_hbm.at[s, u], src_v)
                pltpu.sync_copy(dst2_hbm.at[s, u], dst_v)
                pltpu.sync_copy(w2_hbm.at[s, u], w_v)

            gather_start(i32(0), rows_a, sem_ga)

            def pair(p, carry):
                a = p * i32(2)

                @pl.when(p > i32(0))
                def _():
                    scatter_wait(a - i32(1), rows_b, sem_sb)

                gather_start(a + i32(1), rows_b, sem_gb)
                gather_wait(a, rows_a, sem_ga)
                scale(a, rows_a)
                scatter_start(a, rows_a, sem_sa)
                gather_wait(a + i32(1), rows_b, sem_gb)
                scale(a + i32(1), rows_b)
                scatter_wait(a, rows_a, sem_sa)
                gather_start(a + i32(2), rows_a, sem_ga)
                scatter_start(a + i32(1), rows_b, sem_sb)
                return carry

            lax.fori_loop(i32(0), i32(SUP // 2), pair, i32(0))
            last = i32(SUP - 1)
            gather_wait(last, rows_a, sem_ga)
            scale(last, rows_a)
            scatter_wait(last - i32(1), rows_b, sem_sb)
            scatter_start(last, rows_a, sem_sa)
            scatter_wait(last, rows_a, sem_sa)
            return carry0

        lax.fori_loop(i32(0), i32(NSUP), superchunk, i32(0))
        plsc.subcore_barrier()
        pltpu.sync_copy(acc.at[pl.ds(s * i32(RPT), RPT)],
                        out_hbm.at[c, pl.ds(s * i32(RPT), RPT)])

    return spmm


_spmm = _make_spmm()


BLK = 1024

_PREC = lax.Precision.HIGHEST


def _dot(a, b):
    return jnp.dot(a, b, preferred_element_type=jnp.float32, precision=_PREC)


def _tc_layer_body(p_ref, w_ref, lw1_ref, lb1_ref, lw2_ref, lb2_ref,
                   lw3_ref, lb3_ref, sacc_ref, y_ref, s_ref):
    h = p_ref[0]
    r = jnp.maximum(h, 0.0)
    nrm = jnp.sqrt(jnp.sum(r * r, axis=1, keepdims=True))
    x = r / jnp.maximum(nrm, 1e-12)
    y_ref[0] = _dot(x, w_ref[...])
    m = jnp.maximum(_dot(x, lw1_ref[...]) + lb1_ref[...], 0.0)
    m = jnp.maximum(_dot(m, lw2_ref[...]) + lb2_ref[...], 0.0)
    s_ref[0] = sacc_ref[0] + _dot(m, lw3_ref[...]) + lb3_ref[...]


def _tc_tail_body(p_ref, lw1_ref, lb1_ref, lw2_ref, lb2_ref,
                  lw3_ref, lb3_ref, sacc_ref, s_ref):
    def head(b):
        x = jnp.maximum(p_ref[b], 0.0)
        m = jnp.maximum(_dot(x, lw1_ref[...]) + lb1_ref[...], 0.0)
        m = jnp.maximum(_dot(m, lw2_ref[...]) + lb2_ref[...], 0.0)
        return sacc_ref[b] + _dot(m, lw3_ref[...]) + lb3_ref[...]

    s_ref[...] = head(0) * head(1)


def _full(shape):
    return pl.BlockSpec(shape, lambda *_: tuple(jnp.int32(0) for _ in shape))


def _tc_layer(p, w, lw1, lb1, lw2, lb2, lw3, lb3, sacc):
    return pl.pallas_call(
        _tc_layer_body,
        grid=(NC, NP // BLK),
        in_specs=[
            pl.BlockSpec((1, BLK, NHID), lambda b, i: (b, i, jnp.int32(0))),
            _full((NHID, NHID)),
            _full((NHID, 2 * NHID)), _full((1, 2 * NHID)),
            _full((2 * NHID, 2 * NHID)), _full((1, 2 * NHID)),
            _full((2 * NHID, 1)), _full((1, 1)),
            pl.BlockSpec((1, BLK, 1), lambda b, i: (b, i, jnp.int32(0))),
        ],
        out_specs=[
            pl.BlockSpec((1, BLK, NHID), lambda b, i: (b, i, jnp.int32(0))),
            pl.BlockSpec((1, BLK, 1), lambda b, i: (b, i, jnp.int32(0))),
        ],
        out_shape=[
            jax.ShapeDtypeStruct((NC, NP, NHID), jnp.float32),
            jax.ShapeDtypeStruct((NC, NP, 1), jnp.float32),
        ],
    )(p, w, lw1, lb1, lw2, lb2, lw3, lb3, sacc)


def _tc_tail(p, lw1, lb1, lw2, lb2, lw3, lb3, sacc):
    return pl.pallas_call(
        _tc_tail_body,
        grid=(NP // BLK,),
        in_specs=[
            pl.BlockSpec((NC, BLK, NHID), lambda i: (jnp.int32(0), i, jnp.int32(0))),
            _full((NHID, 2 * NHID)), _full((1, 2 * NHID)),
            _full((2 * NHID, 2 * NHID)), _full((1, 2 * NHID)),
            _full((2 * NHID, 1)), _full((1, 1)),
            pl.BlockSpec((NC, BLK, 1), lambda i: (jnp.int32(0), i, jnp.int32(0))),
        ],
        out_specs=pl.BlockSpec((BLK, 1), lambda i: (i, jnp.int32(0))),
        out_shape=jax.ShapeDtypeStruct((NP, 1), jnp.float32),
    )(p, lw1, lb1, lw2, lb2, lw3, lb3, sacc)


def kernel(edge_index1, edge_weight1, edge_index2, edge_weight2,
           W1, W2, W3, W4, lw1, lb1, lw2, lb2, lw3, lb3):
    out_dtype = jnp.result_type(edge_weight1.dtype, W1.dtype, lw1.dtype,
                                lb1.dtype, lw3.dtype)
    f32 = jnp.float32
    W1, W2, W3, W4 = (a.astype(f32) for a in (W1, W2, W3, W4))
    lw1, lw2, lw3 = (a.astype(f32) for a in (lw1, lw2, lw3))
    lb1, lb2, lb3 = (a.astype(f32) for a in (lb1, lb2, lb3))
    zeros = jnp.zeros((RPT, NHID), jnp.float32)
    lb1r = lb1.reshape(1, 2 * NHID)
    lb2r = lb2.reshape(1, 2 * NHID)
    lb3r = lb3.reshape(1, 1)
    s0 = jnp.zeros((NC, NP, 1), jnp.float32)

    def edges(ei, ew):
        i = ei.astype(jnp.int32).reshape(2, NS, NSUP, SUP, CHUNK)
        return i[1], i[0], ew.astype(f32).reshape(NS, NSUP, SUP, CHUNK)

    src1, dst1, w1 = edges(edge_index1, edge_weight1)
    src2, dst2, w2 = edges(edge_index2, edge_weight2)

    x = jnp.stack([W1, W1])
    p = _spmm(src1, dst1, w1, src2, dst2, w2, x, zeros)
    y, s = _tc_layer(p, W2, lw1, lb1r, lw2, lb2r, lw3, lb3r, s0)
    p = _spmm(src1, dst1, w1, src2, dst2, w2, y, zeros)
    y, s = _tc_layer(p, W3, lw1, lb1r, lw2, lb2r, lw3, lb3r, s)
    p = _spmm(src1, dst1, w1, src2, dst2, w2, y, zeros)
    y, s = _tc_layer(p, W4, lw1, lb1r, lw2, lb2r, lw3, lb3r, s)
    p = _spmm(src1, dst1, w1, src2, dst2, w2, y, zeros)
    out = _tc_tail(p, lw1, lb1r, lw2, lb2r, lw3, lb3r, s)
    return out[:N].astype(out_dtype)


# speedup vs baseline: 261.5734x; 1.1818x over previous
import functools

import jax
import jax.numpy as jnp
from jax import lax
from jax.experimental import pallas as pl
from jax.experimental.pallas import tpu as pltpu
from jax.experimental.pallas import tpu_sc as plsc

N = 10000
NP = 10240
E = 320000
NHID = 128

NC = 2
NS = 16
NW = NC * NS
CHUNK = 80
SUP = 25
NSUP = E // (NW * CHUNK * SUP)
RPT = NP // NS


def _make_spmm():
    mesh = plsc.VectorSubcoreMesh(core_axis_name="c", subcore_axis_name="s")

    @functools.partial(
        pl.kernel,
        out_type=jax.ShapeDtypeStruct((NC, NP, NHID), jnp.float32),
        mesh=mesh,
        scratch_types=[
            pltpu.VMEM((SUP, CHUNK), jnp.int32),
            pltpu.VMEM((SUP, CHUNK), jnp.int32),
            pltpu.VMEM((SUP, CHUNK), jnp.float32),
            pltpu.VMEM((CHUNK, NHID), jnp.float32),
            pltpu.VMEM((CHUNK, NHID), jnp.float32),
            pltpu.VMEM_SHARED((NP, NHID), jnp.float32),
            pltpu.SemaphoreType.DMA,
            pltpu.SemaphoreType.DMA,
            pltpu.SemaphoreType.DMA,
            pltpu.SemaphoreType.DMA,
        ],
    )
    def spmm(src_hbm, dst_hbm, w_hbm, x_hbm, zeros_hbm, out_hbm,
             src_v, dst_v, w_v, rows_a, rows_b, acc,
             sem_ga, sem_gb, sem_sa, sem_sb):
        i32 = jnp.int32
        c = lax.axis_index("c").astype(i32)
        s = lax.axis_index("s").astype(i32)
        wid = c * i32(NS) + s

        pltpu.sync_copy(zeros_hbm, acc.at[pl.ds(s * i32(RPT), RPT)])
        plsc.subcore_barrier()

        def gather_start(j, buf, sem):
            pltpu.async_copy(x_hbm.at[src_v.at[j]], buf, sem)

        def gather_wait(j, buf, sem):
            pltpu.make_async_copy(x_hbm.at[src_v.at[j]], buf, sem).wait()

        def scatter_start(j, buf, sem):
            pltpu.async_copy(buf, acc.at[dst_v.at[j]], sem, add=True)

        def scatter_wait(j, buf, sem):
            pltpu.make_async_copy(buf, acc.at[dst_v.at[j]], sem).wait()

        def scale(j, buf):
            def scale16(g, carry2):
                w16 = w_v[j, pl.ds(g * i32(16), 16)]
                for r in range(16):
                    wr = w16[r]
                    row = g * i32(16) + i32(r)
                    for cj in range(NHID // 16):
                        sl = pl.ds(cj * 16, 16)
                        buf[row, sl] = buf[row, sl] * wr
                return carry2

            lax.fori_loop(i32(0), i32(CHUNK // 16), scale16, i32(0))

        def superchunk(u, carry0):
            pltpu.sync_copy(src_hbm.at[wid, u], src_v)
            pltpu.sync_copy(dst_hbm.at[wid, u], dst_v)
            pltpu.sync_copy(w_hbm.at[wid, u], w_v)

            gather_start(i32(0), rows_a, sem_ga)

            def pair(p, carry):
                a = p * i32(2)

                @pl.when(p > i32(0))
                def _():
                    scatter_wait(a - i32(1), rows_b, sem_sb)

                gather_start(a + i32(1), rows_b, sem_gb)
                gather_wait(a, rows_a, sem_ga)
                scale(a, rows_a)
                scatter_start(a, rows_a, sem_sa)
                gather_wait(a + i32(1), rows_b, sem_gb)
                scale(a + i32(1), rows_b)
                scatter_wait(a, rows_a, sem_sa)
                gather_start(a + i32(2), rows_a, sem_ga)
                scatter_start(a + i32(1), rows_b, sem_sb)
                return carry

            lax.fori_loop(i32(0), i32(SUP // 2), pair, i32(0))
            last = i32(SUP - 1)
            gather_wait(last, rows_a, sem_ga)
            scale(last, rows_a)
            scatter_wait(last - i32(1), rows_b, sem_sb)
            scatter_start(last, rows_a, sem_sa)
            scatter_wait(last, rows_a, sem_sa)
            return carry0

        lax.fori_loop(i32(0), i32(NSUP), superchunk, i32(0))
        plsc.subcore_barrier()
        pltpu.sync_copy(acc.at[pl.ds(s * i32(RPT), RPT)],
                        out_hbm.at[c, pl.ds(s * i32(RPT), RPT)])

    return spmm


_spmm = _make_spmm()


BLK = 1024


def _dot3(a, b):
    f32 = jnp.float32
    bf = jnp.bfloat16
    ah = a.astype(bf)
    al = (a - ah.astype(f32)).astype(bf)
    bh = b.astype(bf)
    bl = (b - bh.astype(f32)).astype(bf)

    def d(u, v):
        return jnp.dot(u, v, preferred_element_type=f32)

    return d(ah, bh) + (d(ah, bl) + d(al, bh))


def _tc_layer_body(p_ref, w_ref, lw1_ref, lb1_ref, lw2_ref, lb2_ref,
                   lw3_ref, lb3_ref, sacc_ref, y_ref, s_ref):
    h = p_ref[0] + p_ref[1]
    r = jnp.maximum(h, 0.0)
    nrm = jnp.sqrt(jnp.sum(r * r, axis=1, keepdims=True))
    x = r / jnp.maximum(nrm, 1e-12)
    y_ref[...] = _dot3(x, w_ref[...])
    m = jnp.maximum(_dot3(x, lw1_ref[...]) + lb1_ref[...], 0.0)
    m = jnp.maximum(_dot3(m, lw2_ref[...]) + lb2_ref[...], 0.0)
    s_ref[...] = sacc_ref[...] + _dot3(m, lw3_ref[...]) + lb3_ref[...]


def _tc_tail_body(p_ref, lw1_ref, lb1_ref, lw2_ref, lb2_ref,
                  lw3_ref, lb3_ref, sacc_ref, mul_ref, s_ref):
    h = p_ref[0] + p_ref[1]
    x = jnp.maximum(h, 0.0)
    m = jnp.maximum(_dot3(x, lw1_ref[...]) + lb1_ref[...], 0.0)
    m = jnp.maximum(_dot3(m, lw2_ref[...]) + lb2_ref[...], 0.0)
    s = sacc_ref[...] + _dot3(m, lw3_ref[...]) + lb3_ref[...]
    s_ref[...] = s * mul_ref[...]


def _full(shape):
    return pl.BlockSpec(shape, lambda *_: tuple(jnp.int32(0) for _ in shape))


def _tc_layer(p, w, lw1, lb1, lw2, lb2, lw3, lb3, sacc):
    return pl.pallas_call(
        _tc_layer_body,
        grid=(NP // BLK,),
        in_specs=[
            pl.BlockSpec((NC, BLK, NHID),
                         lambda i: (jnp.int32(0), i, jnp.int32(0))),
            _full((NHID, NHID)),
            _full((NHID, 2 * NHID)), _full((1, 2 * NHID)),
            _full((2 * NHID, 2 * NHID)), _full((1, 2 * NHID)),
            _full((2 * NHID, 1)), _full((1, 1)),
            pl.BlockSpec((BLK, 1), lambda i: (i, jnp.int32(0))),
        ],
        out_specs=[
            pl.BlockSpec((BLK, NHID), lambda i: (i, jnp.int32(0))),
            pl.BlockSpec((BLK, 1), lambda i: (i, jnp.int32(0))),
        ],
        out_shape=[
            jax.ShapeDtypeStruct((NP, NHID), jnp.float32),
            jax.ShapeDtypeStruct((NP, 1), jnp.float32),
        ],
    )(p, w, lw1, lb1, lw2, lb2, lw3, lb3, sacc)


def _tc_tail(p, lw1, lb1, lw2, lb2, lw3, lb3, sacc, mul):
    return pl.pallas_call(
        _tc_tail_body,
        grid=(NP // BLK,),
        in_specs=[
            pl.BlockSpec((NC, BLK, NHID),
                         lambda i: (jnp.int32(0), i, jnp.int32(0))),
            _full((NHID, 2 * NHID)), _full((1, 2 * NHID)),
            _full((2 * NHID, 2 * NHID)), _full((1, 2 * NHID)),
            _full((2 * NHID, 1)), _full((1, 1)),
            pl.BlockSpec((BLK, 1), lambda i: (i, jnp.int32(0))),
            pl.BlockSpec((BLK, 1), lambda i: (i, jnp.int32(0))),
        ],
        out_specs=pl.BlockSpec((BLK, 1), lambda i: (i, jnp.int32(0))),
        out_shape=jax.ShapeDtypeStruct((NP, 1), jnp.float32),
    )(p, lw1, lb1, lw2, lb2, lw3, lb3, sacc, mul)


def kernel(edge_index1, edge_weight1, edge_index2, edge_weight2,
           W1, W2, W3, W4, lw1, lb1, lw2, lb2, lw3, lb3):
    out_dtype = jnp.result_type(edge_weight1.dtype, W1.dtype, lw1.dtype,
                                lb1.dtype, lw3.dtype)
    f32 = jnp.float32
    W1, W2, W3, W4 = (a.astype(f32) for a in (W1, W2, W3, W4))
    lw1, lw2, lw3 = (a.astype(f32) for a in (lw1, lw2, lw3))
    lb1, lb2, lb3 = (a.astype(f32) for a in (lb1, lb2, lb3))
    zeros = jnp.zeros((RPT, NHID), jnp.float32)
    lb1r = lb1.reshape(1, 2 * NHID)
    lb2r = lb2.reshape(1, 2 * NHID)
    lb3r = lb3.reshape(1, 1)
    s0 = jnp.zeros((NP, 1), jnp.float32)
    ones = jnp.ones((NP, 1), jnp.float32)

    def edges(ei, ew):
        i = ei.astype(jnp.int32).reshape(2, NW, NSUP, SUP, CHUNK)
        return i[1], i[0], ew.astype(f32).reshape(NW, NSUP, SUP, CHUNK)

    def branch(ei, ew, mul):
        src, dst, w = edges(ei, ew)
        p = _spmm(src, dst, w, W1, zeros)
        y, s = _tc_layer(p, W2, lw1, lb1r, lw2, lb2r, lw3, lb3r, s0)
        p = _spmm(src, dst, w, y, zeros)
        y, s = _tc_layer(p, W3, lw1, lb1r, lw2, lb2r, lw3, lb3r, s)
        p = _spmm(src, dst, w, y, zeros)
        y, s = _tc_layer(p, W4, lw1, lb1r, lw2, lb2r, lw3, lb3r, s)
        p = _spmm(src, dst, w, y, zeros)
        return _tc_tail(p, lw1, lb1r, lw2, lb2r, lw3, lb3r, s, mul)

    s1 = branch(edge_index1, edge_weight1, ones)
    return branch(edge_index2, edge_weight2, s1)[:N].astype(out_dtype)
